# baseline (speedup 1.0000x reference)
import dataclasses
import functools

import numpy as np

import jax
import jax.numpy as jnp
from jax import lax
from jax.experimental import pallas as pl
from jax.experimental.pallas import tpu as pltpu
from jax.experimental.pallas import tpu_sc as plsc

N = 10000
D = 128
E = 320000
NPAD = 10240
NC = 2
NS = 16
L = 16
NW = NC * NS
CHUNK = 80
NBUF = 4
EPT = NPAD
NCHUNK = EPT // CHUNK
EPAD = NW * EPT
RPT = NPAD // NS
NSTAGE = 4
CPS = NCHUNK // NSTAGE
SIGMA = 1.0

_PAD_SRC = np.arange(EPAD - E, dtype=np.int32) % N
_PAD_DST = N + np.arange(EPAD - E, dtype=np.int32) % (NPAD - N)

def _sc_cnt_body(dstg, zcnt, cntp, dst_v, cnt_v):
    c = lax.axis_index("c")
    s = lax.axis_index("s")
    wid = c * NS + s
    pltpu.sync_copy(zcnt, cnt_v)
    ones = jnp.ones((L,), jnp.float32)
    for st in range(NSTAGE):
        pltpu.sync_copy(dstg.at[wid, pl.ds(st * CPS, CPS)], dst_v)

        @pl.loop(0, CPS)
        def _(j):
            @pl.loop(0, CHUNK // L)
            def _(q):
                idx = dst_v[j, pl.ds(q * L, L)]
                plsc.addupdate_scatter(cnt_v, [idx], ones)

    pltpu.sync_copy(cnt_v, cntp.at[wid])


def _sc_agg_body(h_hbm, srcg, dstg, zrows, out,
                 src_v, dst_v, rows_v, acc_sh, *sems):
    c = lax.axis_index("c")
    s = lax.axis_index("s")
    wid = c * NS + s

    pltpu.sync_copy(zrows, rows_v.at[0])
    for k in range(RPT // CHUNK):
        pltpu.sync_copy(rows_v.at[0],
                        acc_sh.at[pl.ds(s * RPT + k * CHUNK, CHUNK)])

    plsc.subcore_barrier()

    def _gather(j, b):
        pltpu.async_copy(h_hbm.at[src_v.at[j]], rows_v.at[b], sems[b])

    def _wait(b):
        pltpu.make_async_copy(h_hbm.at[pl.ds(0, CHUNK)], rows_v.at[b],
                              sems[b]).wait()

    for st in range(NSTAGE):
        pltpu.sync_copy(srcg.at[wid, pl.ds(st * CPS, CPS)], src_v)
        pltpu.sync_copy(dstg.at[wid, pl.ds(st * CPS, CPS)], dst_v)
        for b in range(NBUF - 1):
            _gather(b, b)

        @pl.loop(0, CPS, step=NBUF)
        def _(jj):
            for b in range(NBUF):
                _wait(b)
                nxt = jj + b + NBUF - 1

                @pl.when(nxt < CPS)
                def _():
                    _gather(nxt, (b + NBUF - 1) % NBUF)

                pltpu.sync_copy(rows_v.at[b], acc_sh.at[dst_v.at[jj + b]],
                                add=True)

    plsc.subcore_barrier()

    for k in range(RPT // CHUNK):
        off = s * RPT + k * CHUNK
        pltpu.sync_copy(acc_sh.at[pl.ds(off, CHUNK)], rows_v.at[0])
        pltpu.sync_copy(rows_v.at[0], out.at[c, pl.ds(off, CHUNK)])


@functools.cache
def _sc_kernels():
    mesh = plsc.VectorSubcoreMesh(
        core_axis_name="c", subcore_axis_name="s",
        num_cores=NC, num_subcores=NS)
    scratch = [
        pltpu.VMEM((CPS, CHUNK), jnp.int32),
        pltpu.VMEM((CPS, CHUNK), jnp.int32),
        pltpu.VMEM((NBUF, CHUNK, D), jnp.float32),
        pltpu.VMEM_SHARED((NPAD, D), jnp.float32),
    ] + [pltpu.SemaphoreType.DMA] * NBUF
    cp = pltpu.CompilerParams()
    if "needs_layout_passes" in pltpu.CompilerParams.__dataclass_fields__:
        cp = dataclasses.replace(cp, needs_layout_passes=False)
    cnt = pl.kernel(
        _sc_cnt_body,
        out_type=jax.ShapeDtypeStruct((NW, NPAD), jnp.float32),
        mesh=mesh,
        scratch_types=[
            pltpu.VMEM((CPS, CHUNK), jnp.int32),
            pltpu.VMEM((NPAD,), jnp.float32),
        ],
        compiler_params=cp,
    )
    agg = pl.kernel(
        _sc_agg_body,
        out_type=jax.ShapeDtypeStruct((NC, NPAD, D), jnp.float32),
        mesh=mesh,
        scratch_types=scratch,
        compiler_params=cp,
    )
    return cnt, agg


def _emb_body(x_ref, w_ref, b_ref, o_ref):
    o_ref[...] = (
        jnp.dot(x_ref[...], w_ref[...], preferred_element_type=jnp.float32)
        + b_ref[...]
    )


def _self_body(h_ref, w_ref, b_ref, o_ref):
    o_ref[...] = (
        jnp.dot(h_ref[...], w_ref[0:D], preferred_element_type=jnp.float32)
        + b_ref[...]
    )


def _layer_core(h, hw, tot, rinv, w_ref, g_ref, be_ref):
    c = tot[0:N] * rinv
    out = (
        hw
        + jnp.dot(c, w_ref[D:2 * D], preferred_element_type=jnp.float32)
    )
    nrm = jnp.sqrt(jnp.sum(out * out, axis=1, keepdims=True))
    out = out / jnp.maximum(nrm, 1e-12)
    out = jnp.maximum(out, 0.0)
    mu = jnp.mean(out, axis=0, keepdims=True)
    var = jnp.mean((out - mu) ** 2, axis=0, keepdims=True)
    out = g_ref[...] * (out - mu) / jnp.sqrt(var + 1e-5) + be_ref[...]
    return h + out


def _l0_body(h_ref, hw_ref, parts_ref, cntt_ref, w_ref, g_ref, be_ref,
             o_ref, rinv_ref):
    cnt = jnp.sum(cntt_ref[...], axis=1, keepdims=True)
    rinv = 1.0 / jnp.maximum(cnt, 1.0)
    rinv_ref[...] = rinv
    tot = parts_ref[0] + parts_ref[1]
    o_ref[...] = _layer_core(h_ref[...], hw_ref[...], tot, rinv[0:N],
                             w_ref, g_ref, be_ref)


def _lmid_body(h_ref, hw_ref, parts_ref, rinv_ref, w_ref, g_ref, be_ref,
               o_ref):
    tot = parts_ref[0] + parts_ref[1]
    o_ref[...] = _layer_core(h_ref[...], hw_ref[...], tot, rinv_ref[0:N],
                             w_ref, g_ref, be_ref)


def _l1_body(h_ref, hw_ref, parts_ref, rinv_ref, w_ref, g_ref, be_ref,
             wa_ref, ba_ref, o_ref, s_ref):
    tot = parts_ref[0] + parts_ref[1]
    hn = _layer_core(h_ref[...], hw_ref[...], tot, rinv_ref[0:N],
                     w_ref, g_ref, be_ref)
    o_ref[...] = hn
    z = (jnp.dot(hn, wa_ref[...], preferred_element_type=jnp.float32)
         + ba_ref[...]) / SIGMA
    m = jnp.max(z, axis=-1, keepdims=True)
    ez = jnp.exp(z - m)
    s_ref[...] = ez / jnp.sum(ez, axis=-1, keepdims=True)


def _l3_body(h_ref, hw_ref, parts_ref, rinv_ref, w_ref, g_ref, be_ref,
             w0_ref, b0_ref, w1_ref, b1_ref, w2_ref, b2_ref, logits_ref):
    tot = parts_ref[0] + parts_ref[1]
    hn = _layer_core(h_ref[...], hw_ref[...], tot, rinv_ref[0:N],
                     w_ref, g_ref, be_ref)
    hg = jnp.mean(hn, axis=0, keepdims=True)
    z = jnp.maximum(
        jnp.dot(hg, w0_ref[...], preferred_element_type=jnp.float32)
        + b0_ref[...], 0.0)
    z = jnp.maximum(
        jnp.dot(z, w1_ref[...], preferred_element_type=jnp.float32)
        + b1_ref[...], 0.0)
    logits_ref[...] = (
        jnp.dot(z, w2_ref[...], preferred_element_type=jnp.float32)
        + b2_ref[...])


_f32 = jnp.float32
_emb_tc = pl.pallas_call(
    _emb_body, out_shape=jax.ShapeDtypeStruct((N, D), _f32))
_self_tc = pl.pallas_call(
    _self_body, out_shape=jax.ShapeDtypeStruct((N, D), _f32))
_l0_tc = pl.pallas_call(
    _l0_body,
    out_shape=[jax.ShapeDtypeStruct((N, D), _f32),
               jax.ShapeDtypeStruct((NPAD, 1), _f32)])
_lmid_tc = pl.pallas_call(
    _lmid_body, out_shape=jax.ShapeDtypeStruct((N, D), _f32))
_l1_tc = pl.pallas_call(
    _l1_body,
    out_shape=[jax.ShapeDtypeStruct((N, D), _f32),
               jax.ShapeDtypeStruct((N, 32), _f32)])
_l3_tc = pl.pallas_call(
    _l3_body, out_shape=jax.ShapeDtypeStruct((1, 10), _f32))


def kernel(x, e, edge_index, params):
    del e
    src = edge_index[0].astype(jnp.int32)
    dst = edge_index[1].astype(jnp.int32)
    srcg = jnp.concatenate([src, _PAD_SRC]).reshape(NW, NCHUNK, CHUNK)
    dstg = jnp.concatenate([dst, _PAD_DST]).reshape(NW, NCHUNK, CHUNK)
    zrows = jnp.zeros((CHUNK, D), _f32)
    zcnt = jnp.zeros((NPAD,), _f32)

    p = params

    def r2(v):
        return v.reshape(1, -1)

    _sc_cnt, _sc_agg = _sc_kernels()

    cntp = _sc_cnt(dstg, zcnt)
    h = _emb_tc(x, p['emb']['W'], r2(p['emb']['b']))

    parts = _sc_agg(h, srcg, dstg, zrows)
    hw = _self_tc(h, p['l0']['W'], r2(p['l0']['b']))
    cntt = cntp.T
    h, rinv = _l0_tc(h, hw, parts, cntt, p['l0']['W'],
                     r2(p['l0']['gamma']), r2(p['l0']['beta']))

    parts = _sc_agg(h, srcg, dstg, zrows)
    hw = _self_tc(h, p['l1']['W'], r2(p['l1']['b']))
    h, s = _l1_tc(h, hw, parts, rinv, p['l1']['W'],
                  r2(p['l1']['gamma']), r2(p['l1']['beta']),
                  p['assign']['W'], r2(p['assign']['b']))

    parts = _sc_agg(h, srcg, dstg, zrows)
    hw = _self_tc(h, p['l2']['W'], r2(p['l2']['b']))
    h = _lmid_tc(h, hw, parts, rinv, p['l2']['W'],
                 r2(p['l2']['gamma']), r2(p['l2']['beta']))

    parts = _sc_agg(h, srcg, dstg, zrows)
    hw = _self_tc(h, p['l3']['W'], r2(p['l3']['b']))
    logits = _l3_tc(h, hw, parts, rinv, p['l3']['W'],
                    r2(p['l3']['gamma']), r2(p['l3']['beta']),
                    p['mlp0']['W'], r2(p['mlp0']['b']),
                    p['mlp1']['W'], r2(p['mlp1']['b']),
                    p['mlp2']['W'], r2(p['mlp2']['b']))
    return (logits, s)


# speedup vs baseline: 1.0175x; 1.0175x over previous
import dataclasses
import functools

import numpy as np

import jax
import jax.numpy as jnp
from jax import lax
from jax.experimental import pallas as pl
from jax.experimental.pallas import tpu as pltpu
from jax.experimental.pallas import tpu_sc as plsc

N = 10000
D = 128
E = 320000
NPAD = 10240
NC = 2
NS = 16
L = 16
NW = NC * NS
CHUNK = 80
NBUF = 4
EPT = NPAD
NCHUNK = EPT // CHUNK
EPAD = NW * EPT
RPT = NPAD // NS
NSTAGE = 4
CPS = NCHUNK // NSTAGE
SIGMA = 1.0

_PAD_SRC = np.arange(EPAD - E, dtype=np.int32) % N
_PAD_DST = N + np.arange(EPAD - E, dtype=np.int32) % (NPAD - N)

def _sc_cnt_body(dstg, zcnt, cntp, dst_v, cnt_v):
    c = lax.axis_index("c")
    s = lax.axis_index("s")
    wid = c * NS + s
    pltpu.sync_copy(zcnt, cnt_v)
    ones = jnp.ones((L,), jnp.float32)
    for st in range(NSTAGE):
        pltpu.sync_copy(dstg.at[wid, pl.ds(st * CPS, CPS)], dst_v)

        @pl.loop(0, CPS)
        def _(j):
            @pl.loop(0, CHUNK // L)
            def _(q):
                idx = dst_v[j, pl.ds(q * L, L)]
                plsc.addupdate_scatter(cnt_v, [idx], ones)

    pltpu.sync_copy(cnt_v, cntp.at[wid])


def _sc_agg_body(h_hbm, srcg, dstg, zrows, out,
                 src_v, dst_v, rows_v, acc_sh, *sems):
    c = lax.axis_index("c")
    s = lax.axis_index("s")
    wid = c * NS + s

    pltpu.sync_copy(zrows, rows_v.at[0])
    for k in range(RPT // CHUNK):
        pltpu.sync_copy(rows_v.at[0],
                        acc_sh.at[pl.ds(s * RPT + k * CHUNK, CHUNK)])

    plsc.subcore_barrier()

    def _gather(j, b):
        pltpu.async_copy(h_hbm.at[src_v.at[j]], rows_v.at[b], sems[b])

    def _wait(b):
        pltpu.make_async_copy(h_hbm.at[pl.ds(0, CHUNK)], rows_v.at[b],
                              sems[b]).wait()

    for st in range(NSTAGE):
        pltpu.sync_copy(srcg.at[wid, pl.ds(st * CPS, CPS)], src_v)
        pltpu.sync_copy(dstg.at[wid, pl.ds(st * CPS, CPS)], dst_v)
        for b in range(NBUF - 1):
            _gather(b, b)

        @pl.loop(0, CPS, step=NBUF)
        def _(jj):
            for b in range(NBUF):
                _wait(b)
                nxt = jj + b + NBUF - 1

                @pl.when(nxt < CPS)
                def _():
                    _gather(nxt, (b + NBUF - 1) % NBUF)

                pltpu.sync_copy(rows_v.at[b], acc_sh.at[dst_v.at[jj + b]],
                                add=True)

    plsc.subcore_barrier()

    for k in range(RPT // CHUNK):
        off = s * RPT + k * CHUNK
        pltpu.sync_copy(acc_sh.at[pl.ds(off, CHUNK)], rows_v.at[0])
        pltpu.sync_copy(rows_v.at[0], out.at[c, pl.ds(off, CHUNK)])


@functools.cache
def _sc_kernels():
    mesh = plsc.VectorSubcoreMesh(
        core_axis_name="c", subcore_axis_name="s",
        num_cores=NC, num_subcores=NS)
    scratch = [
        pltpu.VMEM((CPS, CHUNK), jnp.int32),
        pltpu.VMEM((CPS, CHUNK), jnp.int32),
        pltpu.VMEM((NBUF, CHUNK, D), jnp.float32),
        pltpu.VMEM_SHARED((NPAD, D), jnp.float32),
    ] + [pltpu.SemaphoreType.DMA] * NBUF
    cp = pltpu.CompilerParams()
    if "needs_layout_passes" in pltpu.CompilerParams.__dataclass_fields__:
        cp = dataclasses.replace(cp, needs_layout_passes=False)
    cnt = pl.kernel(
        _sc_cnt_body,
        out_type=jax.ShapeDtypeStruct((NW, NPAD), jnp.float32),
        mesh=mesh,
        scratch_types=[
            pltpu.VMEM((CPS, CHUNK), jnp.int32),
            pltpu.VMEM((NPAD,), jnp.float32),
        ],
        compiler_params=cp,
    )
    agg = pl.kernel(
        _sc_agg_body,
        out_type=jax.ShapeDtypeStruct((NC, NPAD, D), jnp.float32),
        mesh=mesh,
        scratch_types=scratch,
        compiler_params=cp,
    )
    return cnt, agg


def _emb_body(x_ref, w_ref, b_ref, o_ref):
    o_ref[...] = (
        jnp.dot(x_ref[...], w_ref[...], preferred_element_type=jnp.float32)
        + b_ref[...]
    )


def _layer_core(h, tot, rinv, w_ref, b_ref, g_ref, be_ref):
    c = tot[0:N] * rinv
    out = (
        jnp.dot(h, w_ref[0:D], preferred_element_type=jnp.float32)
        + jnp.dot(c, w_ref[D:2 * D], preferred_element_type=jnp.float32)
        + b_ref[...]
    )
    nrm = jnp.sqrt(jnp.sum(out * out, axis=1, keepdims=True))
    out = out / jnp.maximum(nrm, 1e-12)
    out = jnp.maximum(out, 0.0)
    mu = jnp.mean(out, axis=0, keepdims=True)
    var = jnp.mean((out - mu) ** 2, axis=0, keepdims=True)
    out = g_ref[...] * (out - mu) / jnp.sqrt(var + 1e-5) + be_ref[...]
    return h + out


def _l0_body(h_ref, parts_ref, cntt_ref, w_ref, b_ref, g_ref, be_ref,
             o_ref, rinv_ref):
    cnt = jnp.sum(cntt_ref[...], axis=1, keepdims=True)
    rinv = 1.0 / jnp.maximum(cnt, 1.0)
    rinv_ref[...] = rinv
    tot = parts_ref[0] + parts_ref[1]
    o_ref[...] = _layer_core(h_ref[...], tot, rinv[0:N],
                             w_ref, b_ref, g_ref, be_ref)


def _lmid_body(h_ref, parts_ref, rinv_ref, w_ref, b_ref, g_ref, be_ref,
               o_ref):
    tot = parts_ref[0] + parts_ref[1]
    o_ref[...] = _layer_core(h_ref[...], tot, rinv_ref[0:N],
                             w_ref, b_ref, g_ref, be_ref)


def _l1_body(h_ref, parts_ref, rinv_ref, w_ref, b_ref, g_ref, be_ref,
             wa_ref, ba_ref, o_ref, s_ref):
    tot = parts_ref[0] + parts_ref[1]
    hn = _layer_core(h_ref[...], tot, rinv_ref[0:N],
                     w_ref, b_ref, g_ref, be_ref)
    o_ref[...] = hn
    z = (jnp.dot(hn, wa_ref[...], preferred_element_type=jnp.float32)
         + ba_ref[...]) / SIGMA
    m = jnp.max(z, axis=-1, keepdims=True)
    ez = jnp.exp(z - m)
    s_ref[...] = ez / jnp.sum(ez, axis=-1, keepdims=True)


def _l3_body(h_ref, parts_ref, rinv_ref, w_ref, b_ref, g_ref, be_ref,
             w0_ref, b0_ref, w1_ref, b1_ref, w2_ref, b2_ref, logits_ref):
    tot = parts_ref[0] + parts_ref[1]
    hn = _layer_core(h_ref[...], tot, rinv_ref[0:N],
                     w_ref, b_ref, g_ref, be_ref)
    hg = jnp.mean(hn, axis=0, keepdims=True)
    z = jnp.maximum(
        jnp.dot(hg, w0_ref[...], preferred_element_type=jnp.float32)
        + b0_ref[...], 0.0)
    z = jnp.maximum(
        jnp.dot(z, w1_ref[...], preferred_element_type=jnp.float32)
        + b1_ref[...], 0.0)
    logits_ref[...] = (
        jnp.dot(z, w2_ref[...], preferred_element_type=jnp.float32)
        + b2_ref[...])


_f32 = jnp.float32
_emb_tc = pl.pallas_call(
    _emb_body, out_shape=jax.ShapeDtypeStruct((N, D), _f32))
_l0_tc = pl.pallas_call(
    _l0_body,
    out_shape=[jax.ShapeDtypeStruct((N, D), _f32),
               jax.ShapeDtypeStruct((NPAD, 1), _f32)])
_lmid_tc = pl.pallas_call(
    _lmid_body, out_shape=jax.ShapeDtypeStruct((N, D), _f32))
_l1_tc = pl.pallas_call(
    _l1_body,
    out_shape=[jax.ShapeDtypeStruct((N, D), _f32),
               jax.ShapeDtypeStruct((N, 32), _f32)])
_l3_tc = pl.pallas_call(
    _l3_body, out_shape=jax.ShapeDtypeStruct((1, 10), _f32))


def kernel(x, e, edge_index, params):
    del e
    src = edge_index[0].astype(jnp.int32)
    dst = edge_index[1].astype(jnp.int32)
    srcg = jnp.concatenate([src, _PAD_SRC]).reshape(NW, NCHUNK, CHUNK)
    dstg = jnp.concatenate([dst, _PAD_DST]).reshape(NW, NCHUNK, CHUNK)
    zrows = jnp.zeros((CHUNK, D), _f32)
    zcnt = jnp.zeros((NPAD,), _f32)

    p = params

    def r2(v):
        return v.reshape(1, -1)

    _sc_cnt, _sc_agg = _sc_kernels()

    cntp = _sc_cnt(dstg, zcnt)
    h = _emb_tc(x, p['emb']['W'], r2(p['emb']['b']))

    parts = _sc_agg(h, srcg, dstg, zrows)
    cntt = cntp.T
    h, rinv = _l0_tc(h, parts, cntt, p['l0']['W'], r2(p['l0']['b']),
                     r2(p['l0']['gamma']), r2(p['l0']['beta']))

    parts = _sc_agg(h, srcg, dstg, zrows)
    h, s = _l1_tc(h, parts, rinv, p['l1']['W'], r2(p['l1']['b']),
                  r2(p['l1']['gamma']), r2(p['l1']['beta']),
                  p['assign']['W'], r2(p['assign']['b']))

    parts = _sc_agg(h, srcg, dstg, zrows)
    h = _lmid_tc(h, parts, rinv, p['l2']['W'], r2(p['l2']['b']),
                 r2(p['l2']['gamma']), r2(p['l2']['beta']))

    parts = _sc_agg(h, srcg, dstg, zrows)
    logits = _l3_tc(h, parts, rinv, p['l3']['W'], r2(p['l3']['b']),
                    r2(p['l3']['gamma']), r2(p['l3']['beta']),
                    p['mlp0']['W'], r2(p['mlp0']['b']),
                    p['mlp1']['W'], r2(p['mlp1']['b']),
                    p['mlp2']['W'], r2(p['mlp2']['b']))
    return (logits, s)


# speedup vs baseline: 1.0187x; 1.0011x over previous
import dataclasses
import functools

import numpy as np

import jax
import jax.numpy as jnp
from jax import lax
from jax.experimental import pallas as pl
from jax.experimental.pallas import tpu as pltpu
from jax.experimental.pallas import tpu_sc as plsc

N = 10000
D = 128
E = 320000
NPAD = 10240
NC = 2
NS = 16
L = 16
NW = NC * NS
CHUNK = 80
NBUF = 4
EPT = NPAD
NCHUNK = EPT // CHUNK
EPAD = NW * EPT
RPT = NPAD // NS
NSTAGE = 4
CPS = NCHUNK // NSTAGE
SIGMA = 1.0

_PAD_SRC = np.arange(EPAD - E, dtype=np.int32) % N
_PAD_DST = N + np.arange(EPAD - E, dtype=np.int32) % (NPAD - N)

def _sc_cnt_body(dstg, zcnt, cntp, dst_v, cnt_v):
    c = lax.axis_index("c")
    s = lax.axis_index("s")
    wid = c * NS + s
    pltpu.sync_copy(zcnt, cnt_v)
    ones = jnp.ones((L,), jnp.float32)
    for st in range(NSTAGE):
        pltpu.sync_copy(dstg.at[wid, pl.ds(st * CPS, CPS)], dst_v)

        @pl.loop(0, CPS)
        def _(j):
            @pl.loop(0, CHUNK // L)
            def _(q):
                idx = dst_v[j, pl.ds(q * L, L)]
                plsc.addupdate_scatter(cnt_v, [idx], ones)

    pltpu.sync_copy(cnt_v, cntp.at[wid])


def _sc_agg_body(h_hbm, srcg, dstg, zrows, out,
                 src_v, dst_v, rows_v, acc_sh, *sems):
    c = lax.axis_index("c")
    s = lax.axis_index("s")
    wid = c * NS + s

    pltpu.sync_copy(zrows, rows_v.at[0])
    for k in range(RPT // CHUNK):
        pltpu.sync_copy(rows_v.at[0],
                        acc_sh.at[pl.ds(s * RPT + k * CHUNK, CHUNK)])

    plsc.subcore_barrier()

    def _gather(j, b):
        pltpu.async_copy(h_hbm.at[src_v.at[pl.ds(j * CHUNK, CHUNK)]],
                         rows_v.at[b], sems[b])

    def _wait(b):
        pltpu.make_async_copy(h_hbm.at[pl.ds(0, CHUNK)], rows_v.at[b],
                              sems[b]).wait()

    for st in range(NSTAGE):
        pltpu.sync_copy(
            srcg.at[pl.ds(wid * EPT + st * CPS * CHUNK, CPS * CHUNK)], src_v)
        pltpu.sync_copy(dstg.at[wid, pl.ds(st * CPS, CPS)], dst_v)
        for b in range(NBUF - 1):
            _gather(b, b)

        @pl.loop(0, CPS, step=NBUF)
        def _(jj):
            for b in range(NBUF):
                _wait(b)
                nxt = jj + b + NBUF - 1

                @pl.when(nxt < CPS)
                def _():
                    _gather(nxt, (b + NBUF - 1) % NBUF)

                pltpu.sync_copy(rows_v.at[b], acc_sh.at[dst_v.at[jj + b]],
                                add=True)

    plsc.subcore_barrier()

    for k in range(RPT // CHUNK):
        off = s * RPT + k * CHUNK
        pltpu.sync_copy(acc_sh.at[pl.ds(off, CHUNK)], rows_v.at[0])
        pltpu.sync_copy(rows_v.at[0], out.at[c, pl.ds(off, CHUNK)])


@functools.cache
def _sc_kernels():
    mesh = plsc.VectorSubcoreMesh(
        core_axis_name="c", subcore_axis_name="s",
        num_cores=NC, num_subcores=NS)
    scratch = [
        pltpu.VMEM((CPS * CHUNK,), jnp.int32),
        pltpu.VMEM((CPS, CHUNK), jnp.int32),
        pltpu.VMEM((NBUF, CHUNK, D), jnp.float32),
        pltpu.VMEM_SHARED((NPAD, D), jnp.float32),
    ] + [pltpu.SemaphoreType.DMA] * NBUF
    cp = pltpu.CompilerParams()
    if "needs_layout_passes" in pltpu.CompilerParams.__dataclass_fields__:
        cp = dataclasses.replace(cp, needs_layout_passes=False)
    cnt = pl.kernel(
        _sc_cnt_body,
        out_type=jax.ShapeDtypeStruct((NW, NPAD), jnp.float32),
        mesh=mesh,
        scratch_types=[
            pltpu.VMEM((CPS, CHUNK), jnp.int32),
            pltpu.VMEM((NPAD,), jnp.float32),
        ],
        compiler_params=cp,
    )
    agg = pl.kernel(
        _sc_agg_body,
        out_type=jax.ShapeDtypeStruct((NC, NPAD, D), jnp.float32),
        mesh=mesh,
        scratch_types=scratch,
        compiler_params=cp,
    )
    return cnt, agg


def _emb_body(x_ref, w_ref, b_ref, o_ref):
    o_ref[...] = (
        jnp.dot(x_ref[...], w_ref[...], preferred_element_type=jnp.float32)
        + b_ref[...]
    )


def _layer_core(h, tot, rinv, w_ref, b_ref, g_ref, be_ref):
    c = tot[0:N] * rinv
    out = (
        jnp.dot(h, w_ref[0:D], preferred_element_type=jnp.float32)
        + jnp.dot(c, w_ref[D:2 * D], preferred_element_type=jnp.float32)
        + b_ref[...]
    )
    nrm = jnp.sqrt(jnp.sum(out * out, axis=1, keepdims=True))
    out = out / jnp.maximum(nrm, 1e-12)
    out = jnp.maximum(out, 0.0)
    mu = jnp.mean(out, axis=0, keepdims=True)
    var = jnp.mean((out - mu) ** 2, axis=0, keepdims=True)
    out = g_ref[...] * (out - mu) / jnp.sqrt(var + 1e-5) + be_ref[...]
    return h + out


def _l0_body(h_ref, parts_ref, cntt_ref, w_ref, b_ref, g_ref, be_ref,
             o_ref, rinv_ref):
    cnt = jnp.sum(cntt_ref[...], axis=1, keepdims=True)
    rinv = 1.0 / jnp.maximum(cnt, 1.0)
    rinv_ref[...] = rinv
    tot = parts_ref[0] + parts_ref[1]
    o_ref[...] = _layer_core(h_ref[...], tot, rinv[0:N],
                             w_ref, b_ref, g_ref, be_ref)


def _lmid_body(h_ref, parts_ref, rinv_ref, w_ref, b_ref, g_ref, be_ref,
               o_ref):
    tot = parts_ref[0] + parts_ref[1]
    o_ref[...] = _layer_core(h_ref[...], tot, rinv_ref[0:N],
                             w_ref, b_ref, g_ref, be_ref)


def _l1_body(h_ref, parts_ref, rinv_ref, w_ref, b_ref, g_ref, be_ref,
             wa_ref, ba_ref, o_ref, s_ref):
    tot = parts_ref[0] + parts_ref[1]
    hn = _layer_core(h_ref[...], tot, rinv_ref[0:N],
                     w_ref, b_ref, g_ref, be_ref)
    o_ref[...] = hn
    z = (jnp.dot(hn, wa_ref[...], preferred_element_type=jnp.float32)
         + ba_ref[...]) / SIGMA
    m = jnp.max(z, axis=-1, keepdims=True)
    ez = jnp.exp(z - m)
    s_ref[...] = ez / jnp.sum(ez, axis=-1, keepdims=True)


def _l3_body(h_ref, parts_ref, rinv_ref, w_ref, b_ref, g_ref, be_ref,
             w0_ref, b0_ref, w1_ref, b1_ref, w2_ref, b2_ref, logits_ref):
    tot = parts_ref[0] + parts_ref[1]
    hn = _layer_core(h_ref[...], tot, rinv_ref[0:N],
                     w_ref, b_ref, g_ref, be_ref)
    hg = jnp.mean(hn, axis=0, keepdims=True)
    z = jnp.maximum(
        jnp.dot(hg, w0_ref[...], preferred_element_type=jnp.float32)
        + b0_ref[...], 0.0)
    z = jnp.maximum(
        jnp.dot(z, w1_ref[...], preferred_element_type=jnp.float32)
        + b1_ref[...], 0.0)
    logits_ref[...] = (
        jnp.dot(z, w2_ref[...], preferred_element_type=jnp.float32)
        + b2_ref[...])


_f32 = jnp.float32
_emb_tc = pl.pallas_call(
    _emb_body, out_shape=jax.ShapeDtypeStruct((N, D), _f32))
_l0_tc = pl.pallas_call(
    _l0_body,
    out_shape=[jax.ShapeDtypeStruct((N, D), _f32),
               jax.ShapeDtypeStruct((NPAD, 1), _f32)])
_lmid_tc = pl.pallas_call(
    _lmid_body, out_shape=jax.ShapeDtypeStruct((N, D), _f32))
_l1_tc = pl.pallas_call(
    _l1_body,
    out_shape=[jax.ShapeDtypeStruct((N, D), _f32),
               jax.ShapeDtypeStruct((N, 32), _f32)])
_l3_tc = pl.pallas_call(
    _l3_body, out_shape=jax.ShapeDtypeStruct((1, 10), _f32))


def kernel(x, e, edge_index, params):
    del e
    src = edge_index[0].astype(jnp.int32)
    dst = edge_index[1].astype(jnp.int32)
    srcg = jnp.concatenate([src, _PAD_SRC])
    dstg = jnp.concatenate([dst, _PAD_DST]).reshape(NW, NCHUNK, CHUNK)
    zrows = jnp.zeros((CHUNK, D), _f32)
    zcnt = jnp.zeros((NPAD,), _f32)

    p = params

    def r2(v):
        return v.reshape(1, -1)

    _sc_cnt, _sc_agg = _sc_kernels()

    cntp = _sc_cnt(dstg, zcnt)
    h = _emb_tc(x, p['emb']['W'], r2(p['emb']['b']))

    parts = _sc_agg(h, srcg, dstg, zrows)
    cntt = cntp.T
    h, rinv = _l0_tc(h, parts, cntt, p['l0']['W'], r2(p['l0']['b']),
                     r2(p['l0']['gamma']), r2(p['l0']['beta']))

    parts = _sc_agg(h, srcg, dstg, zrows)
    h, s = _l1_tc(h, parts, rinv, p['l1']['W'], r2(p['l1']['b']),
                  r2(p['l1']['gamma']), r2(p['l1']['beta']),
                  p['assign']['W'], r2(p['assign']['b']))

    parts = _sc_agg(h, srcg, dstg, zrows)
    h = _lmid_tc(h, parts, rinv, p['l2']['W'], r2(p['l2']['b']),
                 r2(p['l2']['gamma']), r2(p['l2']['beta']))

    parts = _sc_agg(h, srcg, dstg, zrows)
    logits = _l3_tc(h, parts, rinv, p['l3']['W'], r2(p['l3']['b']),
                    r2(p['l3']['gamma']), r2(p['l3']['beta']),
                    p['mlp0']['W'], r2(p['mlp0']['b']),
                    p['mlp1']['W'], r2(p['mlp1']['b']),
                    p['mlp2']['W'], r2(p['mlp2']['b']))
    return (logits, s)
